# baseline (speedup 1.0000x reference)
import functools

import jax
import jax.numpy as jnp
from jax import lax
from jax.experimental import pallas as pl
from jax.experimental.pallas import tpu as pltpu
from jax.experimental.pallas import tpu_sc as plsc

_EPS = 1e-12
_NC = 2
_NS = 16
_NW = _NC * _NS
_CHUNK = 80
_TCBLK = 1600


def _make_sc_gather(n_tokens, emb):
    per_w = n_tokens // _NW
    n_chunks = per_w // _CHUNK
    n4 = n_chunks // 4
    mesh = plsc.VectorSubcoreMesh(core_axis_name="c", subcore_axis_name="s")

    @functools.partial(
        pl.kernel,
        mesh=mesh,
        compiler_params=pltpu.CompilerParams(needs_layout_passes=False),
        out_type=jax.ShapeDtypeStruct((n_tokens, emb), jnp.float32),
        scratch_types=[
            pltpu.VMEM((n_chunks, _CHUNK), jnp.int32),
            pltpu.VMEM((_CHUNK, emb), jnp.float32),
            pltpu.VMEM((_CHUNK, emb), jnp.float32),
            pltpu.VMEM((_CHUNK, emb), jnp.float32),
            pltpu.VMEM((_CHUNK, emb), jnp.float32),
            pltpu.SemaphoreType.DMA,
            pltpu.SemaphoreType.DMA,
            pltpu.SemaphoreType.DMA,
            pltpu.SemaphoreType.DMA,
            pltpu.SemaphoreType.DMA,
            pltpu.SemaphoreType.DMA,
            pltpu.SemaphoreType.DMA,
            pltpu.SemaphoreType.DMA,
        ],
    )
    def sc_kernel(wid_hbm, word_hbm, out_hbm, ids_v,
                  row0, row1, row2, row3,
                  sw0, sw1, sw2, sw3, so0, so1, so2, so3):
        wid = lax.axis_index("s") * _NC + lax.axis_index("c")
        base = wid * per_w
        pltpu.sync_copy(wid_hbm.at[wid], ids_v)

        rows = (row0, row1, row2, row3)
        sws = (sw0, sw1, sw2, sw3)
        sos = (so0, so1, so2, so3)

        def start_gather(ci, b):
            pltpu.make_async_copy(
                word_hbm.at[ids_v.at[ci]], rows[b], sws[b]).start()

        def wait_gather(ci, b):
            pltpu.make_async_copy(
                word_hbm.at[ids_v.at[ci]], rows[b], sws[b]).wait()

        def start_writeback(ci, b):
            pltpu.make_async_copy(
                rows[b], out_hbm.at[pl.ds(base + ci * _CHUNK, _CHUNK)],
                sos[b]).start()

        def wait_writeback(b):
            pltpu.make_async_copy(
                rows[b], out_hbm.at[pl.ds(base, _CHUNK)], sos[b]).wait()

        start_gather(0, 0)
        start_gather(1, 1)
        start_gather(2, 2)

        def loop_body(ci4, carry):
            for u in range(4):
                ci = ci4 * 4 + u
                b = u
                b3 = (u + 3) % 4
                wait_gather(ci, b)
                start_writeback(ci, b)

                @pl.when(ci + 3 < n_chunks)
                def _():
                    @pl.when(ci >= 1)
                    def _():
                        wait_writeback(b3)

                    start_gather(ci + 3, b3)
            return carry

        lax.fori_loop(0, n4, loop_body, 0)
        for b in range(4):
            wait_writeback(b)

    return sc_kernel


def _tc_ln_body(x_ref, pos_ref, ttf_ref, cst_ref, o_ref):
    x = (x_ref[...] + pos_ref[...]
         + ttf_ref[...] * cst_ref[0, :][None, :])
    mean = jnp.mean(x, axis=1, keepdims=True)
    var = jnp.mean(x * x, axis=1, keepdims=True) - mean * mean
    inv = lax.rsqrt(var + _EPS)
    o_ref[...] = ((x - mean) * inv * cst_ref[1, :][None, :]
                  + cst_ref[2, :][None, :])


def _tc_ln(rows, posfull, ttf, cst, n_tokens, emb):
    grid = (n_tokens // _TCBLK,)
    return pl.pallas_call(
        _tc_ln_body,
        grid=grid,
        in_specs=[
            pl.BlockSpec((_TCBLK, emb), lambda b: (b, 0)),
            pl.BlockSpec((_TCBLK, emb), lambda b: (0, 0)),
            pl.BlockSpec((_TCBLK, 1), lambda b: (b, 0)),
            pl.BlockSpec((3, emb), lambda b: (0, 0)),
        ],
        out_specs=pl.BlockSpec((_TCBLK, emb), lambda b: (b, 0)),
        out_shape=jax.ShapeDtypeStruct((n_tokens, emb), jnp.float32),
    )(rows, posfull, ttf, cst)


@jax.jit
def kernel(input_ids, token_type_ids, word_embeddings, position_embeddings,
           token_type_embeddings, ln_gamma, ln_beta):
    bsz, seq = input_ids.shape
    vocab, emb = word_embeddings.shape
    n_tokens = bsz * seq
    per_w = n_tokens // _NW
    n_chunks = per_w // _CHUNK

    ids = input_ids.astype(jnp.int32).reshape(_NW, n_chunks, _CHUNK)
    pos2 = position_embeddings[:seq] + token_type_embeddings[0][None, :]
    posfull = jnp.tile(pos2, (_TCBLK // seq, 1))
    ttf = token_type_ids.astype(jnp.float32).reshape(n_tokens, 1)
    cst = jnp.stack(
        [token_type_embeddings[1] - token_type_embeddings[0],
         ln_gamma, ln_beta])

    sc = _make_sc_gather(n_tokens, emb)
    rows = sc(ids, word_embeddings)
    out = _tc_ln(rows, posfull, ttf, cst, n_tokens, emb)
    return out.reshape(bsz, seq, emb)
